# Initial kernel scaffold; baseline (speedup 1.0000x reference)
import jax
import jax.numpy as jnp
from jax.experimental import pallas as pl


def kernel(input_ids, positions, embed_tokens, ln1_w, ln2_w, wqkv, wo, gate_w, w1, w3, w2, norm_w):
    raise NotImplementedError("write your pallas kernel here")


# speedup vs baseline: 1.4109x; 1.4109x over previous
import functools
import math

import jax
import jax.numpy as jnp
from jax import lax
from jax.experimental import pallas as pl
from jax.experimental.pallas import tpu as pltpu
from jax.experimental.pallas import tpu_sc as plsc

V = 50304; D = 1024; NH = 16; NKV = 4; HD = 64; E = 64; TOPK = 2; FF = 512; L = 2; T = 64
EPS = 1e-06; EMB_MULT = 12.0; RES_MULT = 0.22; ATTN_MULT = 0.015625; THETA = 10000.0
HALF = HD // 2
QKV_COLS = (NH + 2 * NKV) * HD


def _sc_gather_body(table_hbm, idx_hbm, out_hbm, idx_v, rows_v, sem):
    wid = lax.axis_index("s") * 2 + lax.axis_index("c")
    rows_per_w = 8

    @pl.when(wid < T // rows_per_w)
    def _():
        base = wid * rows_per_w
        pltpu.sync_copy(idx_hbm.at[pl.ds(base, rows_per_w)], idx_v)
        pltpu.async_copy(table_hbm.at[idx_v], rows_v, sem).wait()
        pltpu.sync_copy(rows_v, out_hbm.at[pl.ds(base, rows_per_w)])


def _sc_embed_gather(table, ids):
    mesh = plsc.VectorSubcoreMesh(core_axis_name="c", subcore_axis_name="s")
    fn = pl.kernel(
        _sc_gather_body,
        out_type=jax.ShapeDtypeStruct((T, D), jnp.float32),
        mesh=mesh,
        scratch_types=[
            pltpu.VMEM((8,), jnp.int32),
            pltpu.VMEM((8, D), jnp.float32),
            pltpu.SemaphoreType.DMA,
        ],
    )
    return fn(table, ids)


def _attn_body(in_scale, h_ref, ln_ref, wqkv_ref, wo_ref, cos_ref, sin_ref, o_ref):
    res = h_ref[...] * in_scale
    x = res * lax.rsqrt(jnp.mean(res * res, axis=-1, keepdims=True) + EPS) * ln_ref[...]
    qkv = jnp.dot(x, wqkv_ref[...], preferred_element_type=jnp.float32)
    cos = cos_ref[...]
    sin = sin_ref[...]

    def rope(xh):
        x1 = xh[:, :HALF]
        x2 = xh[:, HALF:]
        return jnp.concatenate([x1 * cos - x2 * sin, x1 * sin + x2 * cos], axis=1)

    row = lax.broadcasted_iota(jnp.int32, (T, T), 0)
    col = lax.broadcasted_iota(jnp.int32, (T, T), 1)
    causal = row >= col

    kbase = NH * HD
    vbase = (NH + NKV) * HD
    ks = []
    vs = []
    for kv in range(NKV):
        ks.append(rope(qkv[:, kbase + kv * HD: kbase + (kv + 1) * HD]))
        vs.append(qkv[:, vbase + kv * HD: vbase + (kv + 1) * HD])

    outs = []
    for h in range(NH):
        q = rope(qkv[:, h * HD: (h + 1) * HD])
        k = ks[h // (NH // NKV)]
        v = vs[h // (NH // NKV)]
        s = lax.dot_general(q, k, (((1,), (1,)), ((), ())),
                            preferred_element_type=jnp.float32) * ATTN_MULT
        s = jnp.where(causal, s, -1e30)
        s = s - jnp.max(s, axis=-1, keepdims=True)
        p = jnp.exp(s)
        p = p / jnp.sum(p, axis=-1, keepdims=True)
        outs.append(jnp.dot(p, v, preferred_element_type=jnp.float32))

    o = jnp.concatenate(outs, axis=1)
    o_ref[...] = res + jnp.dot(o, wo_ref[...], preferred_element_type=jnp.float32) * RES_MULT


def _attn_block(h, ln_w, wqkv, wo, cos, sin, in_scale):
    return pl.pallas_call(
        functools.partial(_attn_body, in_scale),
        out_shape=jax.ShapeDtypeStruct((T, D), jnp.float32),
    )(h, ln_w.reshape(1, D), wqkv, wo, cos, sin)


def _router_body(h_ref, ln_ref, gate_ref, xn_ref, coef_ref):
    h = h_ref[...]
    x = h * lax.rsqrt(jnp.mean(h * h, axis=-1, keepdims=True) + EPS) * ln_ref[...]
    xn_ref[...] = x
    logits = jnp.dot(x, gate_ref[...], preferred_element_type=jnp.float32)
    logits = logits - jnp.max(logits, axis=-1, keepdims=True)
    ex = jnp.exp(logits)
    probs = ex / jnp.sum(ex, axis=-1, keepdims=True)

    eiota = lax.broadcasted_iota(jnp.int32, (T, E), 1)
    m1 = jnp.max(probs, axis=-1, keepdims=True)
    i1 = jnp.min(jnp.where(probs == m1, eiota, E), axis=-1, keepdims=True)
    p2 = jnp.where(eiota == i1, -1.0, probs)
    m2 = jnp.max(p2, axis=-1, keepdims=True)
    i2 = jnp.min(jnp.where(p2 == m2, eiota, E), axis=-1, keepdims=True)
    denom = m1 + m2
    coef_ref[...] = (jnp.where(eiota == i1, m1, 0.0)
                     + jnp.where(eiota == i2, m2, 0.0)) / denom


def _router(h, ln_w, gate_w):
    return pl.pallas_call(
        _router_body,
        out_shape=(
            jax.ShapeDtypeStruct((T, D), jnp.float32),
            jax.ShapeDtypeStruct((T, E), jnp.float32),
        ),
    )(h, ln_w.reshape(1, D), gate_w)


def _moe_body(final_norm, ids_ref, xn_ref, coef_ref, res_ref, normw_ref,
              w1_ref, w3_ref, w2_ref, o_ref):
    i = pl.program_id(0)
    x = xn_ref[...]
    h1 = jnp.dot(x, w1_ref[0], preferred_element_type=jnp.float32)
    h3 = jnp.dot(x, w3_ref[0], preferred_element_type=jnp.float32)
    g = (h1 / (1.0 + jnp.exp(-h1))) * h3
    c = coef_ref[pl.ds(i, 1), :]
    g = g * jnp.transpose(c)
    y = jnp.dot(g, w2_ref[0], preferred_element_type=jnp.float32)

    @pl.when(i == 0)
    def _():
        o_ref[...] = res_ref[...] + y * RES_MULT

    @pl.when(i > 0)
    def _():
        o_ref[...] += y * RES_MULT

    if final_norm:
        @pl.when(i == E - 1)
        def _():
            out = o_ref[...]
            o_ref[...] = (out * lax.rsqrt(jnp.mean(out * out, axis=-1, keepdims=True) + EPS)
                          * normw_ref[...])


def _moe(xn, coef_sorted, res, ids, w1, w3, w2, norm_w, final_norm):
    grid_spec = pltpu.PrefetchScalarGridSpec(
        num_scalar_prefetch=1,
        grid=(E,),
        in_specs=[
            pl.BlockSpec((T, D), lambda i, ids: (0, 0)),
            pl.BlockSpec((E, T), lambda i, ids: (0, 0)),
            pl.BlockSpec((T, D), lambda i, ids: (0, 0)),
            pl.BlockSpec((1, D), lambda i, ids: (0, 0)),
            pl.BlockSpec((1, D, FF), lambda i, ids: (ids[i], 0, 0)),
            pl.BlockSpec((1, D, FF), lambda i, ids: (ids[i], 0, 0)),
            pl.BlockSpec((1, FF, D), lambda i, ids: (ids[i], 0, 0)),
        ],
        out_specs=pl.BlockSpec((T, D), lambda i, ids: (0, 0)),
    )
    return pl.pallas_call(
        functools.partial(_moe_body, final_norm),
        grid_spec=grid_spec,
        out_shape=jax.ShapeDtypeStruct((T, D), jnp.float32),
    )(ids, xn, coef_sorted, res, norm_w.reshape(1, D), w1, w3, w2)


def _compact_experts(coef):
    active = jnp.any(coef > 0.0, axis=0)
    order = jnp.argsort(jnp.logical_not(active), stable=True).astype(jnp.int32)
    num_active = jnp.sum(active.astype(jnp.int32))
    last = order[jnp.maximum(num_active - 1, 0)]
    steps = jnp.arange(E, dtype=jnp.int32)
    ids = jnp.where(steps < num_active, order, last)
    coef_rows = jnp.where((steps < num_active)[:, None], coef.T[ids], 0.0)
    return ids, coef_rows


def kernel(input_ids, positions, embed_tokens, ln1_w, ln2_w, wqkv, wo,
           gate_w, w1, w3, w2, norm_w):
    ids = input_ids.astype(jnp.int32)
    h = _sc_embed_gather(embed_tokens, ids)

    posf = positions.astype(jnp.float32)
    inv = jnp.exp(jnp.arange(HALF, dtype=jnp.float32) * (-math.log(THETA) / HALF))
    freqs = posf[:, None] * inv[None, :]
    cos = jnp.cos(freqs)
    sin = jnp.sin(freqs)

    for l in range(L):
        in_scale = EMB_MULT if l == 0 else 1.0
        h = _attn_block(h, ln1_w[l], wqkv[l], wo[l], cos, sin, in_scale)
        xn, coef = _router(h, ln2_w[l], gate_w[l])
        eids, coef_rows = _compact_experts(coef)
        h = _moe(xn, coef_rows, h, eids, w1[l], w3[l], w2[l], norm_w,
                 final_norm=(l == L - 1))
    return h


# speedup vs baseline: 1.4392x; 1.0201x over previous
import functools
import math

import jax
import jax.numpy as jnp
from jax import lax
from jax.experimental import pallas as pl
from jax.experimental.pallas import tpu as pltpu
from jax.experimental.pallas import tpu_sc as plsc

V = 50304; D = 1024; NH = 16; NKV = 4; HD = 64; E = 64; TOPK = 2; FF = 512; L = 2; T = 64
EPS = 1e-06; EMB_MULT = 12.0; RES_MULT = 0.22; ATTN_MULT = 0.015625; THETA = 10000.0
HALF = HD // 2
QKV_COLS = (NH + 2 * NKV) * HD


def _sc_gather_body(table_hbm, idx_hbm, out_hbm, idx_v, rows_v, sem):
    wid = lax.axis_index("s") * 2 + lax.axis_index("c")
    rows_per_w = 8

    @pl.when(wid < T // rows_per_w)
    def _():
        base = wid * rows_per_w
        pltpu.sync_copy(idx_hbm.at[pl.ds(base, rows_per_w)], idx_v)
        pltpu.async_copy(table_hbm.at[idx_v], rows_v, sem).wait()
        pltpu.sync_copy(rows_v, out_hbm.at[pl.ds(base, rows_per_w)])


def _sc_embed_gather(table, ids):
    mesh = plsc.VectorSubcoreMesh(core_axis_name="c", subcore_axis_name="s")
    fn = pl.kernel(
        _sc_gather_body,
        out_type=jax.ShapeDtypeStruct((T, D), jnp.float32),
        mesh=mesh,
        scratch_types=[
            pltpu.VMEM((8,), jnp.int32),
            pltpu.VMEM((8, D), jnp.float32),
            pltpu.SemaphoreType.DMA,
        ],
    )
    return fn(table, ids)


def _attn_body(in_scale, h_ref, ln_ref, wqkv_ref, wo_ref, cos_ref, sin_ref, o_ref):
    res = h_ref[...] * in_scale
    x = res * lax.rsqrt(jnp.mean(res * res, axis=-1, keepdims=True) + EPS) * ln_ref[...]
    qkv = jnp.dot(x, wqkv_ref[...], preferred_element_type=jnp.float32)
    cos = cos_ref[...]
    sin = sin_ref[...]

    def rope(xh):
        x1 = xh[:, :HALF]
        x2 = xh[:, HALF:]
        return jnp.concatenate([x1 * cos - x2 * sin, x1 * sin + x2 * cos], axis=1)

    row = lax.broadcasted_iota(jnp.int32, (T, T), 0)
    col = lax.broadcasted_iota(jnp.int32, (T, T), 1)
    causal = row >= col

    kbase = NH * HD
    vbase = (NH + NKV) * HD
    ks = []
    vs = []
    for kv in range(NKV):
        ks.append(rope(qkv[:, kbase + kv * HD: kbase + (kv + 1) * HD]))
        vs.append(qkv[:, vbase + kv * HD: vbase + (kv + 1) * HD])

    outs = []
    for h in range(NH):
        q = rope(qkv[:, h * HD: (h + 1) * HD])
        k = ks[h // (NH // NKV)]
        v = vs[h // (NH // NKV)]
        s = lax.dot_general(q, k, (((1,), (1,)), ((), ())),
                            preferred_element_type=jnp.float32) * ATTN_MULT
        s = jnp.where(causal, s, -1e30)
        s = s - jnp.max(s, axis=-1, keepdims=True)
        p = jnp.exp(s)
        p = p / jnp.sum(p, axis=-1, keepdims=True)
        outs.append(jnp.dot(p, v, preferred_element_type=jnp.float32))

    o = jnp.concatenate(outs, axis=1)
    o_ref[...] = res + jnp.dot(o, wo_ref[...], preferred_element_type=jnp.float32) * RES_MULT


def _attn_block(h, ln_w, wqkv, wo, cos, sin, in_scale):
    return pl.pallas_call(
        functools.partial(_attn_body, in_scale),
        out_shape=jax.ShapeDtypeStruct((T, D), jnp.float32),
    )(h, ln_w.reshape(1, D), wqkv, wo, cos, sin)


def _router_body(h_ref, ln_ref, gate_ref, xn_ref, coef_ref, ids_ref):
    h = h_ref[...]
    x = h * lax.rsqrt(jnp.mean(h * h, axis=-1, keepdims=True) + EPS) * ln_ref[...]
    xn_ref[...] = x
    logits = jnp.dot(x, gate_ref[...], preferred_element_type=jnp.float32)
    logits = logits - jnp.max(logits, axis=-1, keepdims=True)
    ex = jnp.exp(logits)
    probs = ex / jnp.sum(ex, axis=-1, keepdims=True)

    eiota = lax.broadcasted_iota(jnp.int32, (T, E), 1)
    m1 = jnp.max(probs, axis=-1, keepdims=True)
    i1 = jnp.min(jnp.where(probs == m1, eiota, E), axis=-1, keepdims=True)
    p2 = jnp.where(eiota == i1, -1.0, probs)
    m2 = jnp.max(p2, axis=-1, keepdims=True)
    i2 = jnp.min(jnp.where(p2 == m2, eiota, E), axis=-1, keepdims=True)
    denom = m1 + m2
    coef_te = (jnp.where(eiota == i1, m1, 0.0)
               + jnp.where(eiota == i2, m2, 0.0)) / denom

    sel = (eiota == i1) | (eiota == i2)
    active = jnp.max(jnp.where(sel, 1.0, 0.0), axis=0, keepdims=True)
    er_i = lax.broadcasted_iota(jnp.int32, (E, E), 0)
    ec_i = lax.broadcasted_iota(jnp.int32, (E, E), 1)
    er = er_i.astype(jnp.float32)
    ec = ec_i.astype(jnp.float32)
    lt = jnp.where(er_i <= ec_i, 1.0, 0.0)
    cs = jnp.dot(active, lt, preferred_element_type=jnp.float32)
    num_active = cs[:, E - 1:E]
    pos = cs - active
    onehot = jnp.where((er == pos) & (active == 1.0), 1.0, 0.0)
    ids_act = jnp.max(jnp.where(onehot == 1.0, ec, -1.0), axis=1, keepdims=True)
    eiota_row = lax.broadcasted_iota(jnp.int32, (1, E), 1).astype(jnp.float32)
    last = jnp.max(jnp.where(active == 1.0, eiota_row, -1.0), axis=1, keepdims=True)
    stepcol = lax.broadcasted_iota(jnp.int32, (E, 1), 0).astype(jnp.float32)
    ids_ref[...] = jnp.where(stepcol < num_active, ids_act, last)
    coef_ref[...] = lax.dot_general(onehot, coef_te, (((1,), (1,)), ((), ())),
                                    preferred_element_type=jnp.float32)


def _router(h, ln_w, gate_w):
    return pl.pallas_call(
        _router_body,
        out_shape=(
            jax.ShapeDtypeStruct((T, D), jnp.float32),
            jax.ShapeDtypeStruct((E, T), jnp.float32),
            jax.ShapeDtypeStruct((E, 1), jnp.float32),
        ),
    )(h, ln_w.reshape(1, D), gate_w)


def _moe_body(final_norm, ids_ref, xn_ref, coef_ref, res_ref, normw_ref,
              w1_ref, w3_ref, w2_ref, o_ref):
    i = pl.program_id(0)
    x = xn_ref[...]
    h1 = jnp.dot(x, w1_ref[0], preferred_element_type=jnp.float32)
    h3 = jnp.dot(x, w3_ref[0], preferred_element_type=jnp.float32)
    g = (h1 / (1.0 + jnp.exp(-h1))) * h3
    c = coef_ref[pl.ds(i, 1), :]
    g = g * jnp.transpose(c)
    y = jnp.dot(g, w2_ref[0], preferred_element_type=jnp.float32)

    @pl.when(i == 0)
    def _():
        o_ref[...] = res_ref[...] + y * RES_MULT

    @pl.when(i > 0)
    def _():
        o_ref[...] += y * RES_MULT

    if final_norm:
        @pl.when(i == E - 1)
        def _():
            out = o_ref[...]
            o_ref[...] = (out * lax.rsqrt(jnp.mean(out * out, axis=-1, keepdims=True) + EPS)
                          * normw_ref[...])


def _moe(xn, coef_sorted, res, ids, w1, w3, w2, norm_w, final_norm):
    grid_spec = pltpu.PrefetchScalarGridSpec(
        num_scalar_prefetch=1,
        grid=(E,),
        in_specs=[
            pl.BlockSpec((T, D), lambda i, ids: (0, 0)),
            pl.BlockSpec((E, T), lambda i, ids: (0, 0)),
            pl.BlockSpec((T, D), lambda i, ids: (0, 0)),
            pl.BlockSpec((1, D), lambda i, ids: (0, 0)),
            pl.BlockSpec((1, D, FF), lambda i, ids: (ids[i], 0, 0)),
            pl.BlockSpec((1, D, FF), lambda i, ids: (ids[i], 0, 0)),
            pl.BlockSpec((1, FF, D), lambda i, ids: (ids[i], 0, 0)),
        ],
        out_specs=pl.BlockSpec((T, D), lambda i, ids: (0, 0)),
    )
    return pl.pallas_call(
        functools.partial(_moe_body, final_norm),
        grid_spec=grid_spec,
        out_shape=jax.ShapeDtypeStruct((T, D), jnp.float32),
    )(ids, xn, coef_sorted, res, norm_w.reshape(1, D), w1, w3, w2)


def kernel(input_ids, positions, embed_tokens, ln1_w, ln2_w, wqkv, wo,
           gate_w, w1, w3, w2, norm_w):
    ids = input_ids.astype(jnp.int32)
    h = _sc_embed_gather(embed_tokens, ids)

    posf = positions.astype(jnp.float32)
    inv = jnp.exp(jnp.arange(HALF, dtype=jnp.float32) * (-math.log(THETA) / HALF))
    freqs = posf[:, None] * inv[None, :]
    cos = jnp.cos(freqs)
    sin = jnp.sin(freqs)

    for l in range(L):
        in_scale = EMB_MULT if l == 0 else 1.0
        h = _attn_block(h, ln1_w[l], wqkv[l], wo[l], cos, sin, in_scale)
        xn, coef_rows, ids_f = _router(h, ln2_w[l], gate_w[l])
        eids = ids_f.astype(jnp.int32).reshape(E)
        h = _moe(xn, coef_rows, h, eids, w1[l], w3[l], w2[l], norm_w,
                 final_norm=(l == L - 1))
    return h
